# baseline (speedup 1.0000x reference)
import functools

import jax
import jax.numpy as jnp
import numpy as np
from jax import lax
from jax.experimental import pallas as pl
from jax.experimental.pallas import tpu as pltpu
from jax.experimental.pallas import tpu_sc as plsc

NUM_USERS = 50000
NN = 100000
DD = 64
CB = 16
NCB = DD // CB
E_RAW = 1600000
SUB = 128
K = 640
KJ = K // SUB
NSC = 16
CHPW = 160
EPAD = NSC * CHPW * K
TOTCH = EPAD // K
NNP = 100096
ROWS_PW = NNP // NSC
ZR = 368

_sc_mesh = plsc.VectorSubcoreMesh(core_axis_name="c", subcore_axis_name="s")
_GDN = lax.GatherDimensionNumbers(offset_dims=(), collapsed_slice_dims=(0,),
                                  start_index_map=(0,))


@functools.partial(
    pl.kernel,
    out_type=jax.ShapeDtypeStruct((NCB, NNP, CB), jnp.float32),
    mesh=_sc_mesh,
    scratch_types=[
        pltpu.VMEM((4, 2, KJ, SUB), jnp.int32),
        pltpu.VMEM((4, KJ, SUB), jnp.float32),
        pltpu.VMEM((2, K, CB), jnp.float32),
        pltpu.VMEM_SHARED((NNP, CB), jnp.float32),
        pltpu.SemaphoreType.DMA,
        pltpu.SemaphoreType.DMA,
        pltpu.SemaphoreType.DMA,
        pltpu.SemaphoreType.DMA,
        pltpu.SemaphoreType.DMA,
        pltpu.SemaphoreType.DMA,
    ],
    compiler_params=pltpu.CompilerParams(use_tc_tiling_on_sc=False),
)
def _spmm_sc(e4_hbm, meta_hbm, vals_hbm, out_hbm, meta_v, vals_v,
             g_v, acc, sem0, sem1, sem2, sem3, sem4, sem5):
    cid = lax.axis_index("c")
    sid = lax.axis_index("s")
    sems = (sem0, sem1)
    msems = (sem2, sem3)
    ssems = (sem4, sem5)

    for cb in range(2):
        cbg = cid * 2 + cb

        def _fm(ci, mb):
            pltpu.async_copy(meta_hbm.at[sid * CHPW + ci], meta_v.at[mb],
                             msems[mb % 2])
            pltpu.async_copy(vals_hbm.at[sid * CHPW + ci], vals_v.at[mb],
                             msems[mb % 2])

        def _fg(mb, gb):
            pltpu.make_async_copy(meta_hbm.at[0], meta_v.at[mb],
                                  msems[mb % 2]).wait()
            pltpu.make_async_copy(vals_hbm.at[0], vals_v.at[mb],
                                  msems[mb % 2]).wait()
            src = e4_hbm.at[pl.ds(cbg, NN * NCB - NCB + 1)]
            for j in range(KJ):
                pltpu.async_copy(src.at[meta_v.at[mb, 1, j]],
                                 g_v.at[gb, pl.ds(j * SUB, SUB)], sems[gb])

        def _dg(gb):
            pltpu.make_async_copy(e4_hbm.at[pl.ds(0, K)],
                                  g_v.at[gb], sems[gb]).wait()

        def _cs(mb, gb):
            @pl.loop(0, K // CB, unroll=2)
            def _scale(g):
                j = g // (SUB // CB)
                bo = g % (SUB // CB)
                vv = vals_v[mb, j, pl.ds(bo * CB, CB)]
                for l in range(CB):
                    r = g * CB + l
                    bc = lax.gather(vv, jnp.full((CB, 1), l, jnp.int32),
                                    _GDN, (1,),
                                    mode=lax.GatherScatterMode.PROMISE_IN_BOUNDS)
                    g_v[gb, r, :] = g_v[gb, r, :] * bc

            for j in range(KJ):
                pltpu.async_copy(g_v.at[gb, pl.ds(j * SUB, SUB)],
                                 acc.at[meta_v.at[mb, 0, j]], ssems[gb],
                                 add=True)

        def _ds(gb):
            pltpu.make_async_copy(e4_hbm.at[pl.ds(0, K)],
                                  acc.at[pl.ds(0, K)], ssems[gb]).wait()

        _fm(0, 0)
        _fm(1, 1)

        @pl.loop(0, ZR)
        def _zero(i):
            g_v[0, i, :] = jnp.zeros((CB,), jnp.float32)

        for j in range(ROWS_PW // ZR):
            pltpu.sync_copy(g_v.at[0, pl.ds(0, ZR)],
                            acc.at[pl.ds(sid * ROWS_PW + j * ZR, ZR)])
        plsc.subcore_barrier()

        _fg(0, 0)

        @pl.loop(0, CHPW // 4)
        def _quads(q):
            for k in range(4):
                x = 4 * q + k

                @pl.when(x + 2 < CHPW)
                def _():
                    _fm(x + 2, (k + 2) % 4)

                @pl.when(x >= 1)
                def _():
                    _ds((k + 1) % 2)

                @pl.when(x + 1 < CHPW)
                def _():
                    _fg((k + 1) % 4, (k + 1) % 2)

                _dg(k % 2)
                _cs(k % 4, k % 2)

        _ds((CHPW - 1) % 2)
        plsc.subcore_barrier()
        pltpu.sync_copy(acc.at[pl.ds(sid * ROWS_PW, ROWS_PW)],
                        out_hbm.at[cbg].at[pl.ds(sid * ROWS_PW, ROWS_PW)])
        plsc.subcore_barrier()


def _dense_body(r_ref, e_ref, w_ref, o_ref):
    r = r_ref[...]
    x = jnp.concatenate([r, r * e_ref[...]], axis=1)
    z = jnp.dot(x, w_ref[...], preferred_element_type=jnp.float32)
    o_ref[...] = jnp.maximum(z, 0.2 * z)


_BM = 2000


def _dense(R, emb, W12):
    return pl.pallas_call(
        _dense_body,
        grid=(NN // _BM,),
        in_specs=[
            pl.BlockSpec((_BM, DD), lambda i: (i, 0)),
            pl.BlockSpec((_BM, DD), lambda i: (i, 0)),
            pl.BlockSpec((2 * DD, DD), lambda i: (0, 0)),
        ],
        out_specs=pl.BlockSpec((_BM, DD), lambda i: (i, 0)),
        out_shape=jax.ShapeDtypeStruct((NN, DD), jnp.float32),
    )(R, emb, W12)


def _score_body(u0, i0, u1, i1, u2, i2, o_ref):
    s = (jnp.sum(u0[...] * i0[...], axis=1)
         + jnp.sum(u1[...] * i1[...], axis=1)
         + jnp.sum(u2[...] * i2[...], axis=1))
    o_ref[pl.program_id(0), :] = s


def _scores(e0, e1, e2):
    ng = NUM_USERS // _BM
    su = pl.BlockSpec((_BM, DD), lambda i: (i, 0))
    si = pl.BlockSpec((_BM, DD), lambda i: (i + NUM_USERS // _BM, 0))
    out = pl.pallas_call(
        _score_body,
        grid=(ng,),
        in_specs=[su, si, su, si, su, si],
        out_specs=pl.BlockSpec((ng, _BM), lambda i: (0, 0)),
        out_shape=jax.ShapeDtypeStruct((ng, _BM), jnp.float32),
    )(e0, e0, e1, e1, e2, e2)
    return out.reshape(NUM_USERS)


def kernel(user_indices, item_indices, user_table, item_table,
           W1_0, W2_0, W1_1, W2_1,
           adj_row_0, adj_col_0, adj_val_0,
           adj_row_1, adj_col_1, adj_val_1):
    pad_i = jnp.zeros((EPAD - E_RAW,), jnp.int32)
    pad_f = jnp.zeros((EPAD - E_RAW,), jnp.float32)
    rows = jnp.concatenate([adj_row_0, adj_row_1, pad_i]).reshape(TOTCH, KJ, SUB)
    cols = (jnp.concatenate([adj_col_0, adj_col_1, pad_i]) * NCB
            ).reshape(TOTCH, KJ, SUB)
    vals = jnp.concatenate([adj_val_0, adj_val_1, pad_f]).reshape(TOTCH, KJ, SUB)
    meta = jnp.stack([rows, cols], axis=1)

    emb0 = jnp.concatenate([user_table, item_table], axis=0)
    embs = [emb0]
    emb = emb0
    for (W1, W2) in ((W1_0, W2_0), (W1_1, W2_1)):
        r4 = _spmm_sc(emb.reshape(NN * NCB, CB), meta, vals)
        R = r4[:, :NN, :].transpose(1, 0, 2).reshape(NN, DD)
        emb = _dense(R, emb, jnp.concatenate([W1, W2], axis=0))
        embs.append(emb)

    return _scores(*embs)


# speedup vs baseline: 1.0195x; 1.0195x over previous
import functools

import jax
import jax.numpy as jnp
import numpy as np
from jax import lax
from jax.experimental import pallas as pl
from jax.experimental.pallas import tpu as pltpu
from jax.experimental.pallas import tpu_sc as plsc

NUM_USERS = 50000
NN = 100000
DD = 64
CB = 16
NCB = DD // CB
E_RAW = 1600000
SUB = 128
K = 640
KJ = K // SUB
NSC = 16
CHPW = 160
EPAD = NSC * CHPW * K
TOTCH = EPAD // K
NNP = 100096
ROWS_PW = NNP // NSC
ZR = 368

_sc_mesh = plsc.VectorSubcoreMesh(core_axis_name="c", subcore_axis_name="s")
_GDN = lax.GatherDimensionNumbers(offset_dims=(), collapsed_slice_dims=(0,),
                                  start_index_map=(0,))


@functools.partial(
    pl.kernel,
    out_type=jax.ShapeDtypeStruct((NCB, NNP, CB), jnp.float32),
    mesh=_sc_mesh,
    scratch_types=[
        pltpu.VMEM((4, 2, KJ, SUB), jnp.int32),
        pltpu.VMEM((4, KJ, SUB), jnp.float32),
        pltpu.VMEM((2, K, CB), jnp.float32),
        pltpu.VMEM_SHARED((NNP, CB), jnp.float32),
        pltpu.SemaphoreType.DMA,
        pltpu.SemaphoreType.DMA,
        pltpu.SemaphoreType.DMA,
        pltpu.SemaphoreType.DMA,
        pltpu.SemaphoreType.DMA,
        pltpu.SemaphoreType.DMA,
    ],
    compiler_params=pltpu.CompilerParams(use_tc_tiling_on_sc=False),
)
def _spmm_sc(e4_hbm, meta_hbm, vals_hbm, out_hbm, meta_v, vals_v,
             g_v, acc, sem0, sem1, sem2, sem3, sem4, sem5):
    cid = lax.axis_index("c")
    sid = lax.axis_index("s")
    sems = (sem0, sem1)
    msems = (sem2, sem3)
    ssems = (sem4, sem5)

    for cb in range(2):
        cbg = cid * 2 + cb

        def _fm(ci, mb):
            pltpu.async_copy(meta_hbm.at[sid * CHPW + ci], meta_v.at[mb],
                             msems[mb % 2])
            pltpu.async_copy(vals_hbm.at[sid * CHPW + ci], vals_v.at[mb],
                             msems[mb % 2])

        def _fg(mb, gb):
            pltpu.make_async_copy(meta_hbm.at[0], meta_v.at[mb],
                                  msems[mb % 2]).wait()
            pltpu.make_async_copy(vals_hbm.at[0], vals_v.at[mb],
                                  msems[mb % 2]).wait()
            src = e4_hbm.at[pl.ds(cbg, NN * NCB - NCB + 1)]
            for j in range(KJ):
                pltpu.async_copy(src.at[meta_v.at[mb, 1, j]],
                                 g_v.at[gb, pl.ds(j * SUB, SUB)], sems[gb])

        def _dg(gb):
            pltpu.make_async_copy(e4_hbm.at[pl.ds(0, K)],
                                  g_v.at[gb], sems[gb]).wait()

        def _cs(mb, gb):
            @pl.loop(0, K // CB, unroll=2)
            def _scale(g):
                j = g // (SUB // CB)
                bo = g % (SUB // CB)
                vv = vals_v[mb, j, pl.ds(bo * CB, CB)]
                for l in range(CB):
                    r = g * CB + l
                    bc = lax.gather(vv, jnp.full((CB, 1), l, jnp.int32),
                                    _GDN, (1,),
                                    mode=lax.GatherScatterMode.PROMISE_IN_BOUNDS)
                    g_v[gb, r, :] = g_v[gb, r, :] * bc

            for j in range(KJ):
                pltpu.async_copy(g_v.at[gb, pl.ds(j * SUB, SUB)],
                                 acc.at[meta_v.at[mb, 0, j]], ssems[gb],
                                 add=True)

        def _ds(gb):
            pltpu.make_async_copy(e4_hbm.at[pl.ds(0, K)],
                                  acc.at[pl.ds(0, K)], ssems[gb]).wait()

        _fm(0, 0)
        _fm(1, 1)

        @pl.loop(0, ZR)
        def _zero(i):
            g_v[0, i, :] = jnp.zeros((CB,), jnp.float32)

        for j in range(ROWS_PW // ZR):
            pltpu.sync_copy(g_v.at[0, pl.ds(0, ZR)],
                            acc.at[pl.ds(sid * ROWS_PW + j * ZR, ZR)])
        plsc.subcore_barrier()

        _fg(0, 0)

        @pl.loop(0, CHPW // 4)
        def _quads(q):
            for k in range(4):
                x = 4 * q + k

                @pl.when(x + 2 < CHPW)
                def _():
                    _fm(x + 2, (k + 2) % 4)

                @pl.when(x >= 1)
                def _():
                    _ds((k + 1) % 2)

                @pl.when(x + 1 < CHPW)
                def _():
                    _fg((k + 1) % 4, (k + 1) % 2)

                _dg(k % 2)
                _cs(k % 4, k % 2)

        _ds((CHPW - 1) % 2)
        plsc.subcore_barrier()
        pltpu.sync_copy(acc.at[pl.ds(sid * ROWS_PW, ROWS_PW)],
                        out_hbm.at[cbg].at[pl.ds(sid * ROWS_PW, ROWS_PW)])
        plsc.subcore_barrier()


def _dense_body(r_ref, e_ref, w_ref, o_ref):
    r = r_ref[...]
    x = jnp.concatenate([r, r * e_ref[...]], axis=1)
    z = jnp.dot(x, w_ref[...], preferred_element_type=jnp.float32)
    o_ref[...] = jnp.maximum(z, 0.2 * z)


_BM = 2000


def _dense(R, emb, W12):
    return pl.pallas_call(
        _dense_body,
        grid=(NN // _BM,),
        in_specs=[
            pl.BlockSpec((_BM, DD), lambda i: (i, 0)),
            pl.BlockSpec((_BM, DD), lambda i: (i, 0)),
            pl.BlockSpec((2 * DD, DD), lambda i: (0, 0)),
        ],
        out_specs=pl.BlockSpec((_BM, DD), lambda i: (i, 0)),
        out_shape=jax.ShapeDtypeStruct((NN, DD), jnp.float32),
    )(R, emb, W12)


def _dense2_score_body(ru, ri, e1u, e1i, e0u, e0i, w_ref, o_ref):
    w = w_ref[...]
    xu = jnp.concatenate([ru[...], ru[...] * e1u[...]], axis=1)
    zu = jnp.dot(xu, w, preferred_element_type=jnp.float32)
    e2u = jnp.maximum(zu, 0.2 * zu)
    xi = jnp.concatenate([ri[...], ri[...] * e1i[...]], axis=1)
    zi = jnp.dot(xi, w, preferred_element_type=jnp.float32)
    e2i = jnp.maximum(zi, 0.2 * zi)
    s = (jnp.sum(e0u[...] * e0i[...], axis=1)
         + jnp.sum(e1u[...] * e1i[...], axis=1)
         + jnp.sum(e2u * e2i, axis=1))
    o_ref[pl.program_id(0), :] = s


def _dense2_scores(R2, emb1, emb0, W12):
    ng = NUM_USERS // _BM
    su = pl.BlockSpec((_BM, DD), lambda i: (i, 0))
    si = pl.BlockSpec((_BM, DD), lambda i: (i + NUM_USERS // _BM, 0))
    out = pl.pallas_call(
        _dense2_score_body,
        grid=(ng,),
        in_specs=[su, si, su, si, su, si,
                  pl.BlockSpec((2 * DD, DD), lambda i: (0, 0))],
        out_specs=pl.BlockSpec((ng, _BM), lambda i: (0, 0)),
        out_shape=jax.ShapeDtypeStruct((ng, _BM), jnp.float32),
    )(R2, R2, emb1, emb1, emb0, emb0, W12)
    return out.reshape(NUM_USERS)


def kernel(user_indices, item_indices, user_table, item_table,
           W1_0, W2_0, W1_1, W2_1,
           adj_row_0, adj_col_0, adj_val_0,
           adj_row_1, adj_col_1, adj_val_1):
    pad_i = jnp.zeros((EPAD - E_RAW,), jnp.int32)
    pad_f = jnp.zeros((EPAD - E_RAW,), jnp.float32)
    rows = jnp.concatenate([adj_row_0, adj_row_1, pad_i]).reshape(TOTCH, KJ, SUB)
    cols = (jnp.concatenate([adj_col_0, adj_col_1, pad_i]) * NCB
            ).reshape(TOTCH, KJ, SUB)
    vals = jnp.concatenate([adj_val_0, adj_val_1, pad_f]).reshape(TOTCH, KJ, SUB)
    meta = jnp.stack([rows, cols], axis=1)

    emb0 = jnp.concatenate([user_table, item_table], axis=0)
    r4 = _spmm_sc(emb0.reshape(NN * NCB, CB), meta, vals)
    R1 = r4[:, :NN, :].transpose(1, 0, 2).reshape(NN, DD)
    emb1 = _dense(R1, emb0, jnp.concatenate([W1_0, W2_0], axis=0))
    r4 = _spmm_sc(emb1.reshape(NN * NCB, CB), meta, vals)
    R2 = r4[:, :NN, :].transpose(1, 0, 2).reshape(NN, DD)
    return _dense2_scores(R2, emb1, emb0,
                          jnp.concatenate([W1_1, W2_1], axis=0))
